# Initial kernel scaffold; baseline (speedup 1.0000x reference)
import jax
import jax.numpy as jnp
from jax.experimental import pallas as pl


def kernel(inputs, embed):
    raise NotImplementedError("write your pallas kernel here")


# speedup vs baseline: 1.2757x; 1.2757x over previous
import functools

import jax
import jax.numpy as jnp
from jax import lax
from jax.experimental import pallas as pl
from jax.experimental.pallas import tpu as pltpu
from jax.experimental.pallas import tpu_sc as plsc

_N = 8192
_D = 32
_K = 8192
_NT = 256
_NW = 32
_BPW = _N // _NW
_CHUNK = 128
_L = 16


_W = 2048


def _bf16_rtne(v):
    b = lax.bitcast_convert_type(v, jnp.uint32)
    lsb = (b >> jnp.uint32(16)) & jnp.uint32(1)
    rb = (b + jnp.uint32(0x7FFF) + lsb) & jnp.uint32(0xFFFF0000)
    return lax.bitcast_convert_type(rb, jnp.float32)


def _dist_argmin_body(x_ref, e_ref, ind_ref):
    x = x_ref[...]
    e = e_ref[...]
    ab = jnp.dot(x, e, preferred_element_type=jnp.float32)
    xsq = jnp.sum(x * x, axis=1, keepdims=True)
    esq = jnp.sum(e * e, axis=0, keepdims=True)
    dist = (xsq - 2.0 * ab) + esq
    m = None
    idx = None
    for w in range(_K // _W):
        sub = lax.slice_in_dim(dist, w * _W, (w + 1) * _W, axis=1)
        mv = jnp.min(sub, axis=1, keepdims=True)
        cols = lax.broadcasted_iota(jnp.int32, sub.shape, 1) + w * _W
        iw = jnp.min(jnp.where(sub == mv, cols, _K), axis=1, keepdims=True)
        mv_r = _bf16_rtne(mv)
        if m is None:
            m, idx = mv_r, iw
        else:
            upd = mv < m
            m = jnp.where(upd, mv_r, m)
            idx = jnp.where(upd, iw, idx)
    ind_ref[...] = idx


def _dist_argmin(inputs, embed):
    return pl.pallas_call(
        _dist_argmin_body,
        grid=(_N // _NT,),
        in_specs=[
            pl.BlockSpec((_NT, _D), lambda i: (i, 0)),
            pl.BlockSpec((_D, _K), lambda i: (0, 0)),
        ],
        out_specs=pl.BlockSpec((_NT, 1), lambda i: (i, 0)),
        out_shape=jax.ShapeDtypeStruct((_N, 1), jnp.int32),
    )(inputs, embed)


def _gather_hist_body(table_hbm, idx_hbm, zeros_hbm, ones_hbm,
                      out_hbm, counts_hbm,
                      idx_a, idx_b, rows_a, rows_b, ones_v, counts_sp, sem):
    cid = lax.axis_index("c")
    sid = lax.axis_index("s")
    wid = sid * 2 + cid
    base = wid * _BPW
    row0 = wid * 2

    pltpu.sync_copy(idx_hbm.at[row0], idx_a)
    pltpu.sync_copy(idx_hbm.at[row0 + 1], idx_b)
    cpa = pltpu.async_copy(table_hbm.at[idx_a], rows_a, sem)
    cpb = pltpu.async_copy(table_hbm.at[idx_b], rows_b, sem)
    pltpu.sync_copy(ones_hbm, ones_v)
    @pl.when(sid == 0)
    def _zero():
        pltpu.sync_copy(zeros_hbm, counts_sp)

    plsc.subcore_barrier()
    pltpu.sync_copy(ones_v, counts_sp.at[idx_a], add=True)
    pltpu.sync_copy(ones_v, counts_sp.at[idx_b], add=True)
    cpa.wait()
    cpb.wait()
    pltpu.sync_copy(rows_a, out_hbm.at[pl.ds(base, _CHUNK)])
    pltpu.sync_copy(rows_b, out_hbm.at[pl.ds(base + _CHUNK, _CHUNK)])
    plsc.subcore_barrier()

    @pl.when(sid == 0)
    def _flush():
        pltpu.sync_copy(counts_sp, counts_hbm.at[cid])


def _gather_hist(table_t, idx2, zeros_k, ones_c):
    mesh = plsc.VectorSubcoreMesh(core_axis_name="c", subcore_axis_name="s")
    run = functools.partial(
        pl.kernel,
        out_type=[
            jax.ShapeDtypeStruct((_N, _D), jnp.float32),
            jax.ShapeDtypeStruct((2, _K), jnp.float32),
        ],
        mesh=mesh,
        compiler_params=pltpu.CompilerParams(use_tc_tiling_on_sc=False),
        scratch_types=[
            pltpu.VMEM((_CHUNK,), jnp.int32),
            pltpu.VMEM((_CHUNK,), jnp.int32),
            pltpu.VMEM((_CHUNK, _D), jnp.float32),
            pltpu.VMEM((_CHUNK, _D), jnp.float32),
            pltpu.VMEM((_CHUNK,), jnp.float32),
            pltpu.VMEM_SHARED((_K,), jnp.float32),
            pltpu.SemaphoreType.DMA,
        ],
    )(_gather_hist_body)
    return run(table_t, idx2, zeros_k, ones_c)


def _finalize_body(q_ref, x_ref, pc_ref, qout_ref, loss_ref, perp_ref):
    q = q_ref[...]
    x = x_ref[...]
    d = q - x
    qout_ref[...] = x + d
    loss_ref[...] = jnp.sum(d * d, keepdims=True) * (1.0 / (_N * _D))
    counts = jnp.sum(pc_ref[...], axis=0, keepdims=True)
    avg = counts * (1.0 / _N)
    ent = jnp.sum(avg * jnp.log(avg + 1e-10), keepdims=True)
    perp_ref[...] = jnp.exp(-ent)


def _finalize(q_raw, inputs, pcounts):
    return pl.pallas_call(
        _finalize_body,
        out_shape=[
            jax.ShapeDtypeStruct((_N, _D), jnp.float32),
            jax.ShapeDtypeStruct((1, 1), jnp.float32),
            jax.ShapeDtypeStruct((1, 1), jnp.float32),
        ],
    )(q_raw, inputs, pcounts)


def kernel(inputs, embed):
    ind2d = _dist_argmin(inputs, embed)
    ind = ind2d.reshape(_N)
    table_t = embed.T
    idx2 = ind.reshape(_N // _CHUNK, _CHUNK)
    zeros_k = jnp.zeros((_K,), jnp.float32)
    ones_c = jnp.ones((_CHUNK,), jnp.float32)
    q_raw, pcounts = _gather_hist(table_t, idx2, zeros_k, ones_c)
    quantize, loss, perp = _finalize(q_raw, inputs, pcounts)
    return quantize, loss.reshape(()), perp.reshape(()), ind


# speedup vs baseline: 1.3115x; 1.0280x over previous
import functools

import jax
import jax.numpy as jnp
from jax import lax
from jax.experimental import pallas as pl
from jax.experimental.pallas import tpu as pltpu
from jax.experimental.pallas import tpu_sc as plsc

_N = 8192
_D = 32
_K = 8192
_NT = 256
_NW = 32
_BPW = _N // _NW
_CHUNK = 128
_L = 16


_W = 2048


def _bf16_rtne(v):
    b = lax.bitcast_convert_type(v, jnp.uint32)
    lsb = (b >> jnp.uint32(16)) & jnp.uint32(1)
    rb = (b + jnp.uint32(0x7FFF) + lsb) & jnp.uint32(0xFFFF0000)
    return lax.bitcast_convert_type(rb, jnp.float32)


def _dist_argmin_body(x_ref, e_ref, ind_ref, esq_ref):
    x = x_ref[...]
    e = e_ref[...]

    @pl.when(pl.program_id(0) == 0)
    def _prep():
        esq_ref[...] = jnp.sum(e * e, axis=0, keepdims=True)

    ab = jnp.dot(x, e, preferred_element_type=jnp.float32)
    xsq = jnp.sum(x * x, axis=1, keepdims=True)
    esq = esq_ref[...]
    dist = (xsq - 2.0 * ab) + esq
    m = None
    idx = None
    for w in range(_K // _W):
        sub = lax.slice_in_dim(dist, w * _W, (w + 1) * _W, axis=1)
        mv = jnp.min(sub, axis=1, keepdims=True)
        iw = jnp.argmin(sub, axis=1)[:, None] + w * _W
        mv_r = _bf16_rtne(mv)
        if m is None:
            m, idx = mv_r, iw
        else:
            upd = mv < m
            m = jnp.where(upd, mv_r, m)
            idx = jnp.where(upd, iw, idx)
    ind_ref[...] = idx


def _dist_argmin(inputs, embed):
    return pl.pallas_call(
        _dist_argmin_body,
        grid=(_N // _NT,),
        in_specs=[
            pl.BlockSpec((_NT, _D), lambda i: (i, 0)),
            pl.BlockSpec((_D, _K), lambda i: (0, 0)),
        ],
        out_specs=pl.BlockSpec((_NT, 1), lambda i: (i, 0)),
        out_shape=jax.ShapeDtypeStruct((_N, 1), jnp.int32),
        scratch_shapes=[pltpu.VMEM((1, _K), jnp.float32)],
    )(inputs, embed)


def _gather_hist_body(table_hbm, idx_hbm, zeros_hbm, ones_hbm,
                      out_hbm, counts_hbm,
                      idx_a, idx_b, rows_a, rows_b, ones_v, counts_sp, sem):
    cid = lax.axis_index("c")
    sid = lax.axis_index("s")
    wid = sid * 2 + cid
    base = wid * _BPW
    row0 = wid * 2

    pltpu.sync_copy(idx_hbm.at[row0], idx_a)
    pltpu.sync_copy(idx_hbm.at[row0 + 1], idx_b)
    cpa = pltpu.async_copy(table_hbm.at[idx_a], rows_a, sem)
    cpb = pltpu.async_copy(table_hbm.at[idx_b], rows_b, sem)
    pltpu.sync_copy(ones_hbm, ones_v)
    @pl.when(sid == 0)
    def _zero():
        pltpu.sync_copy(zeros_hbm, counts_sp)

    plsc.subcore_barrier()
    pltpu.sync_copy(ones_v, counts_sp.at[idx_a], add=True)
    pltpu.sync_copy(ones_v, counts_sp.at[idx_b], add=True)
    cpa.wait()
    cpb.wait()
    pltpu.sync_copy(rows_a, out_hbm.at[pl.ds(base, _CHUNK)])
    pltpu.sync_copy(rows_b, out_hbm.at[pl.ds(base + _CHUNK, _CHUNK)])
    plsc.subcore_barrier()

    @pl.when(sid == 0)
    def _flush():
        pltpu.sync_copy(counts_sp, counts_hbm.at[cid])


def _gather_hist(table_t, idx2, zeros_k, ones_c):
    mesh = plsc.VectorSubcoreMesh(core_axis_name="c", subcore_axis_name="s")
    run = functools.partial(
        pl.kernel,
        out_type=[
            jax.ShapeDtypeStruct((_N, _D), jnp.float32),
            jax.ShapeDtypeStruct((2, _K), jnp.float32),
        ],
        mesh=mesh,
        compiler_params=pltpu.CompilerParams(use_tc_tiling_on_sc=False),
        scratch_types=[
            pltpu.VMEM((_CHUNK,), jnp.int32),
            pltpu.VMEM((_CHUNK,), jnp.int32),
            pltpu.VMEM((_CHUNK, _D), jnp.float32),
            pltpu.VMEM((_CHUNK, _D), jnp.float32),
            pltpu.VMEM((_CHUNK,), jnp.float32),
            pltpu.VMEM_SHARED((_K,), jnp.float32),
            pltpu.SemaphoreType.DMA,
        ],
    )(_gather_hist_body)
    return run(table_t, idx2, zeros_k, ones_c)


def _finalize_body(q_ref, x_ref, pc_ref, qout_ref, loss_ref, perp_ref):
    q = q_ref[...]
    x = x_ref[...]
    d = q - x
    qout_ref[...] = x + d
    loss_ref[...] = jnp.sum(d * d, keepdims=True) * (1.0 / (_N * _D))
    counts = jnp.sum(pc_ref[...], axis=0, keepdims=True)
    avg = counts * (1.0 / _N)
    ent = jnp.sum(avg * jnp.log(avg + 1e-10), keepdims=True)
    perp_ref[...] = jnp.exp(-ent)


def _finalize(q_raw, inputs, pcounts):
    return pl.pallas_call(
        _finalize_body,
        out_shape=[
            jax.ShapeDtypeStruct((_N, _D), jnp.float32),
            jax.ShapeDtypeStruct((1, 1), jnp.float32),
            jax.ShapeDtypeStruct((1, 1), jnp.float32),
        ],
    )(q_raw, inputs, pcounts)


def kernel(inputs, embed):
    ind2d = _dist_argmin(inputs, embed)
    ind = ind2d.reshape(_N)
    table_t = embed.T
    idx2 = ind.reshape(_N // _CHUNK, _CHUNK)
    zeros_k = jnp.zeros((_K,), jnp.float32)
    ones_c = jnp.ones((_CHUNK,), jnp.float32)
    q_raw, pcounts = _gather_hist(table_t, idx2, zeros_k, ones_c)
    quantize, loss, perp = _finalize(q_raw, inputs, pcounts)
    return quantize, loss.reshape(()), perp.reshape(()), ind
